# Initial kernel scaffold; baseline (speedup 1.0000x reference)
import jax
import jax.numpy as jnp
from jax.experimental import pallas as pl


def kernel(SIDs, PIDs, implicit_PIDs, implicit_lengths, scientist_emb, paper_emb, implicit_emb, scientist_bias, paper_bias):
    raise NotImplementedError("write your pallas kernel here")


# speedup vs baseline: 1.6696x; 1.6696x over previous
import functools
import jax
import jax.numpy as jnp
from jax import lax
from jax.experimental import pallas as pl
from jax.experimental.pallas import tpu as pltpu
from jax.experimental.pallas import tpu_sc as plsc

_MU = 3.5
_W = 0.2
_B = 16384
_H = 50
_D = 32
_NC = 2
_NS = 16
_NW = _NC * _NS
_RPW = _B // _NW
_CPW = _RPW // 2
_NBUF = 2
_L = 16


def _sc_body(impl_idx, sid2d, pid2d, lengths, s_emb, p_emb, i_emb,
             s_bias, p_bias, tbl, out,
             idx_buf, len_buf, sidx, pidx, ps_buf, qp_buf, bs_buf, bp_buf,
             tbuf, rows0, rows1, out_buf, sem0, sem1, sem_misc):
  wid = lax.axis_index("s") * _NC + lax.axis_index("c")
  rbase = wid * _RPW
  cbase = wid * _CPW
  gbase = wid * (_RPW // 128)

  pltpu.sync_copy(sid2d.at[pl.ds(gbase, _RPW // 128), :], sidx)
  pltpu.sync_copy(pid2d.at[pl.ds(gbase, _RPW // 128), :], pidx)

  handles = [
      pltpu.async_copy(impl_idx.at[pl.ds(cbase, _CPW), :], idx_buf, sem_misc),
      pltpu.async_copy(lengths.at[pl.ds(rbase, _RPW)], len_buf, sem_misc),
      pltpu.async_copy(tbl, tbuf, sem_misc),
  ]
  for g in range(_RPW // 128):
    dst = pl.ds(g * 128, 128)
    handles.append(pltpu.async_copy(s_emb.at[sidx.at[g]],
                                    ps_buf.at[dst, :], sem_misc))
    handles.append(pltpu.async_copy(p_emb.at[pidx.at[g]],
                                    qp_buf.at[dst, :], sem_misc))
    handles.append(pltpu.async_copy(s_bias.at[sidx.at[g]],
                                    bs_buf.at[dst], sem_misc))
    handles.append(pltpu.async_copy(p_bias.at[pidx.at[g]],
                                    bp_buf.at[dst], sem_misc))
  for h in handles:
    h.wait()

  pltpu.async_copy(i_emb.at[idx_buf.at[0]], rows0, sem0)
  pltpu.async_copy(i_emb.at[idx_buf.at[1]], rows1, sem1)

  rows_slots = (rows0, rows1)
  sem_slots = (sem0, sem1)
  lane = lax.iota(jnp.int32, _L)
  m2 = lane < 2

  def step(i, carry):
    for b in range(_NBUF):
      c = i * _NBUF + b
      rows, sem = rows_slots[b], sem_slots[b]
      pltpu.make_async_copy(i_emb.at[idx_buf.at[0]], rows, sem).wait()
      idxv = lane + 2 * c
      lens_v = plsc.load_gather(len_buf, [idxv], mask=m2)
      t_v = plsc.load_gather(tbuf, [jnp.clip(lens_v, 0, 63)], mask=m2)
      bs_v = plsc.load_gather(bs_buf, [idxv], mask=m2)
      bp_v = plsc.load_gather(bp_buf, [idxv], mask=m2)
      preds = []
      for rr in range(2):
        r = c * 2 + rr
        ln = lens_v[rr]
        acc0 = jnp.zeros((_L,), jnp.float32)
        acc1 = jnp.zeros((_L,), jnp.float32)
        for j in range(_H):
          m = jnp.where(j < ln, 1.0, 0.0)
          acc0 = acc0 + rows[rr * _H + j, pl.ds(0, _L)] * m
          acc1 = acc1 + rows[rr * _H + j, pl.ds(_L, _L)] * m
        t = t_v[rr]
        ps0 = plsc.load_gather(ps_buf, [jnp.full((_L,), r, jnp.int32), lane])
        ps1 = plsc.load_gather(ps_buf, [jnp.full((_L,), r, jnp.int32),
                                        lane + _L])
        qp0 = plsc.load_gather(qp_buf, [jnp.full((_L,), r, jnp.int32), lane])
        qp1 = plsc.load_gather(qp_buf, [jnp.full((_L,), r, jnp.int32),
                                        lane + _L])
        v0 = (ps0 + t * acc0) * qp0
        v1 = (ps1 + t * acc1) * qp1
        preds.append(jnp.sum(v0 + v1))
      pred_v = _MU + bs_v + bp_v + jnp.where(lane == 0, preds[0], preds[1])
      plsc.store_scatter(out_buf, [idxv], pred_v, mask=m2)
      @pl.when(c + _NBUF < _CPW)
      def _():
        pltpu.async_copy(i_emb.at[idx_buf.at[c + _NBUF]], rows, sem)
    return carry

  lax.fori_loop(0, _CPW // _NBUF, step, 0)
  pltpu.sync_copy(out_buf, out.at[pl.ds(rbase, _RPW)])


@jax.jit
def _run(impl_idx, sid2d, pid2d, lengths, s_emb, p_emb, i_emb,
         s_bias, p_bias, tbl):
  mesh = plsc.VectorSubcoreMesh(core_axis_name="c", subcore_axis_name="s")
  f = pl.kernel(
      functools.partial(_sc_body),
      out_type=jax.ShapeDtypeStruct((_B,), jnp.float32),
      mesh=mesh,
      scratch_types=[
          pltpu.VMEM((_CPW, 2 * _H), jnp.int32),
          pltpu.VMEM((_RPW,), jnp.int32),
          pltpu.VMEM((_RPW // 128, 128), jnp.int32),
          pltpu.VMEM((_RPW // 128, 128), jnp.int32),
          pltpu.VMEM((_RPW, _D), jnp.float32),
          pltpu.VMEM((_RPW, _D), jnp.float32),
          pltpu.VMEM((_RPW,), jnp.float32),
          pltpu.VMEM((_RPW,), jnp.float32),
          pltpu.VMEM((64,), jnp.float32),
          pltpu.VMEM((2 * _H, _D), jnp.float32),
          pltpu.VMEM((2 * _H, _D), jnp.float32),
          pltpu.VMEM((_RPW,), jnp.float32),
          pltpu.SemaphoreType.DMA,
          pltpu.SemaphoreType.DMA,
          pltpu.SemaphoreType.DMA,
      ],
      compiler_params=pltpu.CompilerParams(needs_layout_passes=False,
                                           use_tc_tiling_on_sc=False),
      name="implicit_mf_sc",
  )
  return f(impl_idx, sid2d, pid2d, lengths, s_emb, p_emb, i_emb,
           s_bias, p_bias, tbl)


def kernel(SIDs, PIDs, implicit_PIDs, implicit_lengths, scientist_emb,
           paper_emb, implicit_emb, scientist_bias, paper_bias):
  impl_idx = implicit_PIDs.astype(jnp.int32).reshape(_B // 2, 2 * _H)
  sid2d = SIDs.astype(jnp.int32).reshape(_B // 128, 128)
  pid2d = PIDs.astype(jnp.int32).reshape(_B // 128, 128)
  lengths = implicit_lengths.astype(jnp.int32)
  tbl = _W / jnp.sqrt(jnp.arange(64, dtype=jnp.float32) + 1e-8)
  return _run(impl_idx, sid2d, pid2d, lengths, scientist_emb, paper_emb,
              implicit_emb, scientist_bias.reshape(-1),
              paper_bias.reshape(-1), tbl)


# speedup vs baseline: 1.7615x; 1.0550x over previous
import functools
import jax
import jax.numpy as jnp
from jax import lax
from jax.experimental import pallas as pl
from jax.experimental.pallas import tpu as pltpu
from jax.experimental.pallas import tpu_sc as plsc

_MU = 3.5
_W = 0.2
_B = 16384
_H = 50
_D = 32
_NC = 2
_NS = 16
_NW = _NC * _NS
_RPW = _B // _NW
_CPW = _RPW // 2
_G = 4
_L = 16


def _sc_body(impl_idx, sid2d, pid2d, lengths, s_emb, p_emb, i_emb,
             s_bias, p_bias, tbl, out,
             idx_buf, len_buf, sidx, pidx, ps_buf, qp_buf, bs_buf, bp_buf,
             tbuf, rows0, rows1, out_buf, sem0, sem1, sem_misc):
  wid = lax.axis_index("s") * _NC + lax.axis_index("c")
  rbase = wid * _RPW
  cbase = wid * _CPW
  gbase = wid * (_RPW // 128)

  pltpu.sync_copy(sid2d.at[pl.ds(gbase, _RPW // 128), :], sidx)
  pltpu.sync_copy(pid2d.at[pl.ds(gbase, _RPW // 128), :], pidx)

  handles = [
      pltpu.async_copy(impl_idx.at[pl.ds(cbase, _CPW), :], idx_buf, sem_misc),
      pltpu.async_copy(lengths.at[pl.ds(rbase, _RPW)], len_buf, sem_misc),
      pltpu.async_copy(tbl, tbuf, sem_misc),
  ]
  for g in range(_RPW // 128):
    dst = pl.ds(g * 128, 128)
    handles.append(pltpu.async_copy(s_emb.at[sidx.at[g]],
                                    ps_buf.at[dst, :], sem_misc))
    handles.append(pltpu.async_copy(p_emb.at[pidx.at[g]],
                                    qp_buf.at[dst, :], sem_misc))
    handles.append(pltpu.async_copy(s_bias.at[sidx.at[g]],
                                    bs_buf.at[dst], sem_misc))
    handles.append(pltpu.async_copy(p_bias.at[pidx.at[g]],
                                    bp_buf.at[dst], sem_misc))
  for h in handles:
    h.wait()

  rows_slots = (rows0, rows1)
  sem_slots = (sem0, sem1)
  lane = lax.iota(jnp.int32, _L)
  m2 = lane < 2
  nsl = _CPW // _G

  def fill(slot_load, b):
    rows, sem = rows_slots[b], sem_slots[b]
    for k in range(_G):
      pltpu.async_copy(i_emb.at[idx_buf.at[slot_load * _G + k]],
                       rows.at[pl.ds(k * 2 * _H, 2 * _H), :], sem)

  fill(0, 0)
  fill(1, 1)

  def step(i, carry):
    for b in range(2):
      sl = 2 * i + b
      rows, sem = rows_slots[b], sem_slots[b]
      for k in range(_G):
        pltpu.make_async_copy(
            i_emb.at[idx_buf.at[0]],
            rows.at[pl.ds(k * 2 * _H, 2 * _H), :], sem).wait()
      for k in range(_G):
        c = sl * _G + k
        idxv = lane + 2 * c
        lens_v = plsc.load_gather(len_buf, [idxv], mask=m2)
        t_v = plsc.load_gather(tbuf, [jnp.clip(lens_v, 0, 63)], mask=m2)
        bs_v = plsc.load_gather(bs_buf, [idxv], mask=m2)
        bp_v = plsc.load_gather(bp_buf, [idxv], mask=m2)
        preds = []
        for rr in range(2):
          r = c * 2 + rr
          ln = lens_v[rr]
          base = (k * 2 + rr) * _H

          def jbody(j, a, base=base, ln=ln, rows=rows):
            m = jnp.where(j < ln, 1.0, 0.0)
            return (a[0] + rows[base + j, pl.ds(0, _L)] * m,
                    a[1] + rows[base + j, pl.ds(_L, _L)] * m)

          acc0, acc1 = lax.fori_loop(
              0, _H, jbody,
              (jnp.zeros((_L,), jnp.float32), jnp.zeros((_L,), jnp.float32)),
              unroll=10)
          t = t_v[rr]
          rv = jnp.full((_L,), r, jnp.int32)
          ps0 = plsc.load_gather(ps_buf, [rv, lane])
          ps1 = plsc.load_gather(ps_buf, [rv, lane + _L])
          qp0 = plsc.load_gather(qp_buf, [rv, lane])
          qp1 = plsc.load_gather(qp_buf, [rv, lane + _L])
          v0 = (ps0 + t * acc0) * qp0
          v1 = (ps1 + t * acc1) * qp1
          preds.append(jnp.sum(v0 + v1))
        pred_v = _MU + bs_v + bp_v + jnp.where(lane == 0, preds[0], preds[1])
        plsc.store_scatter(out_buf, [idxv], pred_v, mask=m2)
      @pl.when(sl + 2 < nsl)
      def _():
        fill(sl + 2, b)
    return carry

  lax.fori_loop(0, nsl // 2, step, 0)
  pltpu.sync_copy(out_buf, out.at[pl.ds(rbase, _RPW)])


@jax.jit
def _run(impl_idx, sid2d, pid2d, lengths, s_emb, p_emb, i_emb,
         s_bias, p_bias, tbl):
  mesh = plsc.VectorSubcoreMesh(core_axis_name="c", subcore_axis_name="s")
  f = pl.kernel(
      functools.partial(_sc_body),
      out_type=jax.ShapeDtypeStruct((_B,), jnp.float32),
      mesh=mesh,
      scratch_types=[
          pltpu.VMEM((_CPW, 2 * _H), jnp.int32),
          pltpu.VMEM((_RPW,), jnp.int32),
          pltpu.VMEM((_RPW // 128, 128), jnp.int32),
          pltpu.VMEM((_RPW // 128, 128), jnp.int32),
          pltpu.VMEM((_RPW, _D), jnp.float32),
          pltpu.VMEM((_RPW, _D), jnp.float32),
          pltpu.VMEM((_RPW,), jnp.float32),
          pltpu.VMEM((_RPW,), jnp.float32),
          pltpu.VMEM((64,), jnp.float32),
          pltpu.VMEM((_G * 2 * _H, _D), jnp.float32),
          pltpu.VMEM((_G * 2 * _H, _D), jnp.float32),
          pltpu.VMEM((_RPW,), jnp.float32),
          pltpu.SemaphoreType.DMA,
          pltpu.SemaphoreType.DMA,
          pltpu.SemaphoreType.DMA,
      ],
      compiler_params=pltpu.CompilerParams(needs_layout_passes=False,
                                           use_tc_tiling_on_sc=False),
      name="implicit_mf_sc",
  )
  return f(impl_idx, sid2d, pid2d, lengths, s_emb, p_emb, i_emb,
           s_bias, p_bias, tbl)


def kernel(SIDs, PIDs, implicit_PIDs, implicit_lengths, scientist_emb,
           paper_emb, implicit_emb, scientist_bias, paper_bias):
  impl_idx = implicit_PIDs.astype(jnp.int32).reshape(_B // 2, 2 * _H)
  sid2d = SIDs.astype(jnp.int32).reshape(_B // 128, 128)
  pid2d = PIDs.astype(jnp.int32).reshape(_B // 128, 128)
  lengths = implicit_lengths.astype(jnp.int32)
  tbl = _W / jnp.sqrt(jnp.arange(64, dtype=jnp.float32) + 1e-8)
  return _run(impl_idx, sid2d, pid2d, lengths, scientist_emb, paper_emb,
              implicit_emb, scientist_bias.reshape(-1),
              paper_bias.reshape(-1), tbl)
